# baseline (speedup 1.0000x reference)
import functools

import jax
import jax.numpy as jnp
from jax import lax
from jax.experimental import pallas as pl
from jax.experimental.pallas import tpu as pltpu
from jax.experimental.pallas import tpu_sc as plsc

N = 10000
E = 160000
D = 256
H = 128
NC, NS = 2, 16
KC = 128
EPAD = 163840
CPT = EPAD // (NS * KC)
NB = 2
NQ = 4
KCC = 128
CPT2 = EPAD // (NC * NS * KCC)
NPAD = 16
RPT = 640
LAST = N - (NS - 1) * RPT
ZR = 40

_f32 = jnp.float32
_mesh = plsc.VectorSubcoreMesh(core_axis_name="c", subcore_axis_name="s")
_cp = pltpu.CompilerParams(use_tc_tiling_on_sc=True)


@functools.partial(
    pl.kernel, mesh=_mesh, compiler_params=_cp,
    out_type=jax.ShapeDtypeStruct((NC, N, H), _f32),
    scratch_types=(
        [pltpu.VMEM((KC,), jnp.int32)] * NQ
        + [pltpu.VMEM((KC,), jnp.int32)] * NQ
        + [pltpu.VMEM((KC, H), _f32)] * NB
        + [
            pltpu.VMEM((ZR, H), _f32),
            pltpu.VMEM_SHARED((N + NPAD, H), _f32),
        ]
        + [pltpu.SemaphoreType.DMA] * NQ
        + [pltpu.SemaphoreType.DMA] * NB
        + [pltpu.SemaphoreType.DMA] * NB
    ))
def _agg(table_hbm, src_hbm, dst_hbm, out_hbm, *rest):
  srcb = rest[0:NQ]
  dstb = rest[NQ:2 * NQ]
  rows = rest[2 * NQ:2 * NQ + NB]
  zrows = rest[2 * NQ + NB]
  acc = rest[2 * NQ + NB + 1]
  isem = rest[2 * NQ + NB + 2:2 * NQ + NB + 2 + NQ]
  gsem = rest[2 * NQ + NB + 2 + NQ:2 * NQ + NB + 2 + NQ + NB]
  ssem = rest[2 * NQ + NB + 2 + NQ + NB:]
  cid = lax.axis_index("c")
  sid = lax.axis_index("s")
  z16 = jnp.zeros((1, 16), _f32)

  @pl.loop(0, ZR)
  def _(r):
    @pl.loop(0, H // 16)
    def _(c):
      zrows.at[pl.ds(r, 1), pl.ds(c * 16, 16)][...] = z16

  row0 = sid * RPT

  @pl.when(sid < NS - 1)
  def _():
    for j in range(RPT // ZR):
      pltpu.sync_copy(zrows, acc.at[pl.ds(row0 + j * ZR, ZR)])

  @pl.when(sid == NS - 1)
  def _():
    for j in range(LAST // ZR):
      pltpu.sync_copy(zrows, acc.at[pl.ds(row0 + j * ZR, ZR)])
    pltpu.sync_copy(zrows.at[pl.ds(0, NPAD)], acc.at[pl.ds(N, NPAD)])

  plsc.subcore_barrier()

  table = table_hbm
  src_c = src_hbm.at[cid]
  e0 = sid * CPT * KC

  def idx_fetch(i, q):
    pltpu.async_copy(src_c.at[pl.ds(e0 + i * KC, KC)], srcb[q], isem[q])
    pltpu.async_copy(dst_hbm.at[pl.ds(e0 + i * KC, KC)], dstb[q], isem[q])

  def idx_wait(q):
    pltpu.make_async_copy(src_c.at[pl.ds(e0, KC)], srcb[q], isem[q]).wait()
    pltpu.make_async_copy(dst_hbm.at[pl.ds(e0, KC)], dstb[q], isem[q]).wait()

  def gather_issue(i, q, b):
    pltpu.async_copy(table.at[srcb[q]], rows[b], gsem[b])

  def gather_wait(b):
    pltpu.make_async_copy(table.at[srcb[0]], rows[b], gsem[b]).wait()

  def scat_issue(q, b):
    pltpu.async_copy(rows[b], acc.at[dstb[q]], add=True, sem=ssem[b])

  def scat_wait(b):
    pltpu.make_async_copy(rows[b], acc.at[dstb[0]], ssem[b]).wait()

  idx_fetch(0, 0)
  idx_fetch(1, 1)

  @pl.loop(0, CPT // NQ)
  def _(t):
    for k in range(NQ):
      j = t * NQ + k
      b, q, f = k % NB, k, (k + 2) % NQ

      @pl.when(j >= NB)
      def _():
        scat_wait(b)

      @pl.when(j + 2 < CPT)
      def _():
        idx_fetch(j + 2, f)

      idx_wait(q)
      gather_issue(j, q, b)

      bp, qp = (k - 1) % NB, (k - 1) % NQ

      @pl.when(j >= 1)
      def _():
        gather_wait(bp)
        scat_issue(qp, bp)

  gather_wait((CPT - 1) % NB)
  scat_issue((CPT - 1) % NQ, (CPT - 1) % NB)
  for b in range(NB):
    scat_wait(b)

  plsc.subcore_barrier()

  @pl.when(sid < NS - 1)
  def _():
    pltpu.sync_copy(acc.at[pl.ds(row0, RPT)],
                    out_hbm.at[cid].at[pl.ds(row0, RPT)])

  @pl.when(sid == NS - 1)
  def _():
    pltpu.sync_copy(acc.at[pl.ds(row0, LAST)],
                    out_hbm.at[cid].at[pl.ds(row0, LAST)])


NQ2 = 8


@functools.partial(
    pl.kernel, mesh=_mesh, compiler_params=_cp,
    out_type=jax.ShapeDtypeStruct((NC, N, H), _f32),
    scratch_types=(
        [pltpu.VMEM((KCC,), jnp.int32)] * NQ2
        + [
            pltpu.VMEM((KCC, H), _f32),
            pltpu.VMEM((ZR, H), _f32),
            pltpu.VMEM_SHARED((N + NPAD, H), _f32),
        ]
        + [pltpu.SemaphoreType.DMA] * NQ2
        + [pltpu.SemaphoreType.DMA] * NQ2
    ))
def _count(dst_hbm, out_hbm, *rest):
  dstb = rest[0:NQ2]
  oneb = rest[NQ2]
  zrows = rest[NQ2 + 1]
  cntacc = rest[NQ2 + 2]
  isem = rest[NQ2 + 3:NQ2 + 3 + NQ2]
  ssem = rest[NQ2 + 3 + NQ2:]
  cid = lax.axis_index("c")
  sid = lax.axis_index("s")
  z16 = jnp.zeros((1, 16), _f32)
  one16 = jnp.ones((1, 16), _f32)

  @pl.loop(0, ZR)
  def _(r):
    @pl.loop(0, H // 16)
    def _(c):
      zrows.at[pl.ds(r, 1), pl.ds(c * 16, 16)][...] = z16

  @pl.loop(0, KCC)
  def _(r):
    @pl.loop(0, H // 16)
    def _(c):
      oneb.at[pl.ds(r, 1), pl.ds(c * 16, 16)][...] = one16

  row0 = sid * RPT

  @pl.when(sid < NS - 1)
  def _():
    for j in range(RPT // ZR):
      pltpu.sync_copy(zrows, cntacc.at[pl.ds(row0 + j * ZR, ZR)])

  @pl.when(sid == NS - 1)
  def _():
    for j in range(LAST // ZR):
      pltpu.sync_copy(zrows, cntacc.at[pl.ds(row0 + j * ZR, ZR)])
    pltpu.sync_copy(zrows.at[pl.ds(0, NPAD)], cntacc.at[pl.ds(N, NPAD)])

  plsc.subcore_barrier()

  e0 = (cid * NS + sid) * CPT2 * KCC

  def idx_fetch(i, q):
    pltpu.async_copy(dst_hbm.at[pl.ds(e0 + i * KCC, KCC)], dstb[q], isem[q])

  def idx_wait(q):
    pltpu.make_async_copy(dst_hbm.at[pl.ds(e0, KCC)], dstb[q], isem[q]).wait()

  def scat_issue(q):
    pltpu.async_copy(oneb, cntacc.at[dstb[q]], add=True, sem=ssem[q])

  def scat_wait(q):
    pltpu.make_async_copy(oneb, cntacc.at[dstb[0]], ssem[q]).wait()

  idx_fetch(0, 0)
  idx_fetch(1, 1)

  @pl.loop(0, CPT2 // NQ2)
  def _(t):
    for k in range(NQ2):
      i = t * NQ2 + k
      q, f = k, (k + 2) % NQ2

      @pl.when(i >= 6)
      def _():
        scat_wait(f)

      @pl.when(i + 2 < CPT2)
      def _():
        idx_fetch(i + 2, f)

      idx_wait(q)
      scat_issue(q)

  for x in range(CPT2 - 6, CPT2):
    scat_wait(x % NQ2)

  plsc.subcore_barrier()

  @pl.when(sid < NS - 1)
  def _():
    pltpu.sync_copy(cntacc.at[pl.ds(row0, RPT)],
                    out_hbm.at[cid].at[pl.ds(row0, RPT)])

  @pl.when(sid == NS - 1)
  def _():
    pltpu.sync_copy(cntacc.at[pl.ds(row0, LAST)],
                    out_hbm.at[cid].at[pl.ds(row0, LAST)])


R = 1000
_CT = (((1,), (1,)), ((), ()))


def _linr1_kernel(x_ref, wr_ref, b_ref, out_ref):
  out_ref[...] = (lax.dot_general(x_ref[...], wr_ref[...], _CT,
                                  preferred_element_type=_f32) + b_ref[...])


def _linr2_kernel(x_ref, wr_ref, b_ref, out_ref):
  xmat = jnp.reshape(x_ref[...], (R, D))
  out_ref[...] = (lax.dot_general(xmat, wr_ref[...], _CT,
                                  preferred_element_type=_f32) + b_ref[...])


def _mean_lin(agg_ref, cnt_ref, zr_ref, wl_ref):
  cnt = cnt_ref[0][:, 0:1] + cnt_ref[1][:, 0:1]
  inv = 1.0 / jnp.maximum(cnt, 1.0)
  wl = wl_ref[...]
  z = lax.dot_general(agg_ref[0] * inv, wl[:, :H], _CT,
                      preferred_element_type=_f32)
  z += lax.dot_general(agg_ref[1] * inv, wl[:, H:], _CT,
                       preferred_element_type=_f32)
  return z + zr_ref[...]


def _post1_kernel(agg_ref, cnt_ref, zr_ref, wl_ref, out_ref):
  h = jnp.maximum(_mean_lin(agg_ref, cnt_ref, zr_ref, wl_ref), 0.0)
  out_ref[...] = jnp.reshape(h, (2 * R, H))


def _post2_kernel(agg_ref, cnt_ref, zr_ref, wl_ref, out_ref):
  z = _mean_lin(agg_ref, cnt_ref, zr_ref, wl_ref)
  nrm = jnp.sqrt(jnp.sum(z * z, axis=1, keepdims=True))
  out_ref[...] = z / jnp.maximum(nrm, 1e-12)


_split_spec = pl.BlockSpec((2, R, H), lambda i: (0, i, 0))
_cnt_spec = pl.BlockSpec((2, R, 8), lambda i: (0, i, 0))
_row_spec = pl.BlockSpec((R, D), lambda i: (i, 0))
_pair_spec = pl.BlockSpec((2 * R, H), lambda i: (i, 0))
_w_spec = pl.BlockSpec((D, D), lambda i: (0, 0))
_b_spec = pl.BlockSpec((1, D), lambda i: (0, 0))

_linr1 = pl.pallas_call(
    _linr1_kernel,
    grid=(N // R,),
    in_specs=[_row_spec, _w_spec, _b_spec],
    out_specs=_row_spec,
    out_shape=jax.ShapeDtypeStruct((N, D), _f32),
)

_linr2 = pl.pallas_call(
    _linr2_kernel,
    grid=(N // R,),
    in_specs=[_pair_spec, _w_spec, _b_spec],
    out_specs=_row_spec,
    out_shape=jax.ShapeDtypeStruct((N, D), _f32),
)

_post1 = pl.pallas_call(
    _post1_kernel,
    grid=(N // R,),
    in_specs=[_split_spec, _cnt_spec, _row_spec, _w_spec],
    out_specs=_pair_spec,
    out_shape=jax.ShapeDtypeStruct((2 * N, H), _f32),
)

_post2 = pl.pallas_call(
    _post2_kernel,
    grid=(N // R,),
    in_specs=[_split_spec, _cnt_spec, _row_spec, _w_spec],
    out_specs=_row_spec,
    out_shape=jax.ShapeDtypeStruct((N, D), _f32),
)


def kernel(x, edge_index, W1_l, b1, W1_r, W2_l, b2, W2_r):
  src = edge_index[0]
  dst = edge_index[1]
  pad = jnp.arange(EPAD - E, dtype=jnp.int32)
  srcp = jnp.concatenate([src, pad % N])
  dstp = jnp.concatenate([dst, N + (pad % NPAD)])
  src2 = jnp.stack([2 * srcp, 2 * srcp + 1])
  b1r = b1.reshape(1, D)
  b2r = b2.reshape(1, D)
  cnt = _count(dstp)
  cnt8 = cnt[:, :, :8]
  xv = x.reshape(2 * N, H)
  dstp_dep = dstp + (0.0 * cnt[0, 0, 0]).astype(jnp.int32)
  agg1 = _agg(xv, src2, dstp_dep)
  zr1 = _linr1(x, W1_r, b1r)
  h1v = _post1(agg1, cnt8, zr1, W1_l)
  agg2 = _agg(h1v, src2, dstp)
  zr2 = _linr2(h1v, W2_r, b2r)
  return _post2(agg2, cnt8, zr2, W2_l)


# speedup vs baseline: 1.0082x; 1.0082x over previous
import functools

import jax
import jax.numpy as jnp
from jax import lax
from jax.experimental import pallas as pl
from jax.experimental.pallas import tpu as pltpu
from jax.experimental.pallas import tpu_sc as plsc

N = 10000
E = 160000
D = 256
H = 128
NC, NS = 2, 16
KC = 128
EPAD = 163840
CPT = EPAD // (NS * KC)
NB = 2
NQ = 4
KCC = 128
CPT2 = EPAD // (NC * NS * KCC)
NPAD = 16
RPT = 640
LAST = N - (NS - 1) * RPT
ZR = 40

_f32 = jnp.float32
_mesh = plsc.VectorSubcoreMesh(core_axis_name="c", subcore_axis_name="s")
_cp = pltpu.CompilerParams(use_tc_tiling_on_sc=True)


@functools.partial(
    pl.kernel, mesh=_mesh, compiler_params=_cp,
    out_type=jax.ShapeDtypeStruct((NC, N, H), _f32),
    scratch_types=(
        [pltpu.VMEM((KC,), jnp.int32)] * NQ
        + [pltpu.VMEM((KC,), jnp.int32)] * NQ
        + [pltpu.VMEM((KC, H), _f32)] * NB
        + [
            pltpu.VMEM((ZR, H), _f32),
            pltpu.VMEM_SHARED((N + NPAD, H), _f32),
        ]
        + [pltpu.SemaphoreType.DMA] * NQ
        + [pltpu.SemaphoreType.DMA] * NB
        + [pltpu.SemaphoreType.DMA] * NB
    ))
def _agg(table_hbm, src_hbm, dst_hbm, out_hbm, *rest):
  srcb = rest[0:NQ]
  dstb = rest[NQ:2 * NQ]
  rows = rest[2 * NQ:2 * NQ + NB]
  zrows = rest[2 * NQ + NB]
  acc = rest[2 * NQ + NB + 1]
  isem = rest[2 * NQ + NB + 2:2 * NQ + NB + 2 + NQ]
  gsem = rest[2 * NQ + NB + 2 + NQ:2 * NQ + NB + 2 + NQ + NB]
  ssem = rest[2 * NQ + NB + 2 + NQ + NB:]
  cid = lax.axis_index("c")
  sid = lax.axis_index("s")
  z16 = jnp.zeros((1, 16), _f32)

  @pl.loop(0, ZR)
  def _(r):
    @pl.loop(0, H // 16)
    def _(c):
      zrows.at[pl.ds(r, 1), pl.ds(c * 16, 16)][...] = z16

  row0 = sid * RPT

  @pl.when(sid < NS - 1)
  def _():
    for j in range(RPT // ZR):
      pltpu.sync_copy(zrows, acc.at[pl.ds(row0 + j * ZR, ZR)])

  @pl.when(sid == NS - 1)
  def _():
    for j in range(LAST // ZR):
      pltpu.sync_copy(zrows, acc.at[pl.ds(row0 + j * ZR, ZR)])
    pltpu.sync_copy(zrows.at[pl.ds(0, NPAD)], acc.at[pl.ds(N, NPAD)])

  plsc.subcore_barrier()

  table = table_hbm
  src_c = src_hbm.at[cid]
  e0 = sid * CPT * KC

  def idx_fetch(i, q):
    pltpu.async_copy(src_c.at[pl.ds(e0 + i * KC, KC)], srcb[q], isem[q])
    pltpu.async_copy(dst_hbm.at[pl.ds(e0 + i * KC, KC)], dstb[q], isem[q])

  def idx_wait(q):
    pltpu.make_async_copy(src_c.at[pl.ds(e0, KC)], srcb[q], isem[q]).wait()
    pltpu.make_async_copy(dst_hbm.at[pl.ds(e0, KC)], dstb[q], isem[q]).wait()

  def gather_issue(i, q, b):
    pltpu.async_copy(table.at[srcb[q]], rows[b], gsem[b])

  def gather_wait(b):
    pltpu.make_async_copy(table.at[srcb[0]], rows[b], gsem[b]).wait()

  def scat_issue(q, b):
    pltpu.async_copy(rows[b], acc.at[dstb[q]], add=True, sem=ssem[b])

  def scat_wait(b):
    pltpu.make_async_copy(rows[b], acc.at[dstb[0]], ssem[b]).wait()

  idx_fetch(0, 0)
  idx_fetch(1, 1)

  @pl.loop(0, CPT // NQ)
  def _(t):
    for k in range(NQ):
      j = t * NQ + k
      b, q, f = k % NB, k, (k + 2) % NQ

      @pl.when(j >= NB)
      def _():
        scat_wait(b)

      @pl.when(j + 2 < CPT)
      def _():
        idx_fetch(j + 2, f)

      idx_wait(q)
      gather_issue(j, q, b)

      bp, qp = (k - 1) % NB, (k - 1) % NQ

      @pl.when(j >= 1)
      def _():
        gather_wait(bp)
        scat_issue(qp, bp)

  gather_wait((CPT - 1) % NB)
  scat_issue((CPT - 1) % NQ, (CPT - 1) % NB)
  for b in range(NB):
    scat_wait(b)

  plsc.subcore_barrier()

  @pl.when(sid < NS - 1)
  def _():
    pltpu.sync_copy(acc.at[pl.ds(row0, RPT)],
                    out_hbm.at[cid].at[pl.ds(row0, RPT)])

  @pl.when(sid == NS - 1)
  def _():
    pltpu.sync_copy(acc.at[pl.ds(row0, LAST)],
                    out_hbm.at[cid].at[pl.ds(row0, LAST)])


NQ2 = 8


@functools.partial(
    pl.kernel, mesh=_mesh, compiler_params=_cp,
    out_type=jax.ShapeDtypeStruct((NC, N, H), _f32),
    scratch_types=(
        [pltpu.VMEM((KCC,), jnp.int32)] * NQ2
        + [
            pltpu.VMEM((KCC, H), _f32),
            pltpu.VMEM((ZR, H), _f32),
            pltpu.VMEM_SHARED((N + NPAD, H), _f32),
        ]
        + [pltpu.SemaphoreType.DMA] * NQ2
        + [pltpu.SemaphoreType.DMA] * NQ2
    ))
def _count(dst_hbm, out_hbm, *rest):
  dstb = rest[0:NQ2]
  oneb = rest[NQ2]
  zrows = rest[NQ2 + 1]
  cntacc = rest[NQ2 + 2]
  isem = rest[NQ2 + 3:NQ2 + 3 + NQ2]
  ssem = rest[NQ2 + 3 + NQ2:]
  cid = lax.axis_index("c")
  sid = lax.axis_index("s")
  z16 = jnp.zeros((1, 16), _f32)
  one16 = jnp.ones((1, 16), _f32)

  @pl.loop(0, ZR)
  def _(r):
    @pl.loop(0, H // 16)
    def _(c):
      zrows.at[pl.ds(r, 1), pl.ds(c * 16, 16)][...] = z16

  @pl.loop(0, KCC)
  def _(r):
    @pl.loop(0, H // 16)
    def _(c):
      oneb.at[pl.ds(r, 1), pl.ds(c * 16, 16)][...] = one16

  row0 = sid * RPT

  @pl.when(sid < NS - 1)
  def _():
    for j in range(RPT // ZR):
      pltpu.sync_copy(zrows, cntacc.at[pl.ds(row0 + j * ZR, ZR)])

  @pl.when(sid == NS - 1)
  def _():
    for j in range(LAST // ZR):
      pltpu.sync_copy(zrows, cntacc.at[pl.ds(row0 + j * ZR, ZR)])
    pltpu.sync_copy(zrows.at[pl.ds(0, NPAD)], cntacc.at[pl.ds(N, NPAD)])

  plsc.subcore_barrier()

  e0 = (cid * NS + sid) * CPT2 * KCC

  def idx_fetch(i, q):
    pltpu.async_copy(dst_hbm.at[pl.ds(e0 + i * KCC, KCC)], dstb[q], isem[q])

  def idx_wait(q):
    pltpu.make_async_copy(dst_hbm.at[pl.ds(e0, KCC)], dstb[q], isem[q]).wait()

  def scat_issue(q):
    pltpu.async_copy(oneb, cntacc.at[dstb[q]], add=True, sem=ssem[q])

  def scat_wait(q):
    pltpu.make_async_copy(oneb, cntacc.at[dstb[0]], ssem[q]).wait()

  idx_fetch(0, 0)
  idx_fetch(1, 1)

  @pl.loop(0, CPT2 // NQ2)
  def _(t):
    for k in range(NQ2):
      i = t * NQ2 + k
      q, f = k, (k + 2) % NQ2

      @pl.when(i >= 6)
      def _():
        scat_wait(f)

      @pl.when(i + 2 < CPT2)
      def _():
        idx_fetch(i + 2, f)

      idx_wait(q)
      scat_issue(q)

  for x in range(CPT2 - 6, CPT2):
    scat_wait(x % NQ2)

  plsc.subcore_barrier()

  @pl.when(sid < NS - 1)
  def _():
    pltpu.sync_copy(cntacc.at[pl.ds(row0, RPT)],
                    out_hbm.at[cid].at[pl.ds(row0, RPT)])

  @pl.when(sid == NS - 1)
  def _():
    pltpu.sync_copy(cntacc.at[pl.ds(row0, LAST)],
                    out_hbm.at[cid].at[pl.ds(row0, LAST)])


R = 1000
_CT = (((1,), (1,)), ((), ()))


def _dense_body(agg_ref, cnt_ref, xmat, wl_ref, wr_ref, b_ref):
  cnt = cnt_ref[0][:, 0:1] + cnt_ref[1][:, 0:1]
  inv = 1.0 / jnp.maximum(cnt, 1.0)
  wl = wl_ref[...]
  z = lax.dot_general(agg_ref[0] * inv, wl[:, :H], _CT,
                      preferred_element_type=_f32)
  z += lax.dot_general(agg_ref[1] * inv, wl[:, H:], _CT,
                       preferred_element_type=_f32)
  z += lax.dot_general(xmat, wr_ref[...], _CT, preferred_element_type=_f32)
  return z + b_ref[...]


def _dense1_kernel(agg_ref, cnt_ref, x_ref, wl_ref, wr_ref, b_ref, out_ref):
  h = jnp.maximum(
      _dense_body(agg_ref, cnt_ref, x_ref[...], wl_ref, wr_ref, b_ref), 0.0)
  out_ref[...] = jnp.reshape(h, (2 * R, H))


def _dense2_kernel(agg_ref, cnt_ref, x_ref, wl_ref, wr_ref, b_ref, out_ref):
  xmat = jnp.reshape(x_ref[...], (R, D))
  z = _dense_body(agg_ref, cnt_ref, xmat, wl_ref, wr_ref, b_ref)
  nrm = jnp.sqrt(jnp.sum(z * z, axis=1, keepdims=True))
  out_ref[...] = z / jnp.maximum(nrm, 1e-12)


_split_spec = pl.BlockSpec((2, R, H), lambda i: (0, i, 0))
_cnt_spec = pl.BlockSpec((2, R, 8), lambda i: (0, i, 0))
_row_spec = pl.BlockSpec((R, D), lambda i: (i, 0))
_pair_spec = pl.BlockSpec((2 * R, H), lambda i: (i, 0))
_w_spec = pl.BlockSpec((D, D), lambda i: (0, 0))
_b_spec = pl.BlockSpec((1, D), lambda i: (0, 0))

_dense1 = pl.pallas_call(
    _dense1_kernel,
    grid=(N // R,),
    in_specs=[_split_spec, _cnt_spec, _row_spec, _w_spec, _w_spec, _b_spec],
    out_specs=_pair_spec,
    out_shape=jax.ShapeDtypeStruct((2 * N, H), _f32),
)

_dense2 = pl.pallas_call(
    _dense2_kernel,
    grid=(N // R,),
    in_specs=[_split_spec, _cnt_spec, _pair_spec, _w_spec, _w_spec, _b_spec],
    out_specs=_row_spec,
    out_shape=jax.ShapeDtypeStruct((N, D), _f32),
)


def kernel(x, edge_index, W1_l, b1, W1_r, W2_l, b2, W2_r):
  src = edge_index[0]
  dst = edge_index[1]
  pad = jnp.arange(EPAD - E, dtype=jnp.int32)
  srcp = jnp.concatenate([src, pad % N])
  dstp = jnp.concatenate([dst, N + (pad % NPAD)])
  src2 = jnp.stack([2 * srcp, 2 * srcp + 1])
  b1r = b1.reshape(1, D)
  b2r = b2.reshape(1, D)
  cnt = _count(dstp)
  cnt8 = cnt[:, :, :8]
  xv = x.reshape(2 * N, H)
  dstp_dep = dstp + (0.0 * cnt[0, 0, 0]).astype(jnp.int32)
  agg1 = _agg(xv, src2, dstp_dep)
  h1v = _dense1(agg1, cnt8, x, W1_l, W1_r, b1r)
  agg2 = _agg(h1v, src2, dstp)
  return _dense2(agg2, cnt8, h1v, W2_l, W2_r, b2r)
